# Initial kernel scaffold; baseline (speedup 1.0000x reference)
import jax
import jax.numpy as jnp
from jax.experimental import pallas as pl


def kernel(loc_data, conf_data, priors, gt_bboxes, num_boxes):
    raise NotImplementedError("write your pallas kernel here")


# speedup vs baseline: 4.0589x; 4.0589x over previous
import functools

import jax
import jax.numpy as jnp
from jax.experimental import pallas as pl
from jax.experimental.pallas import tpu as pltpu

_NUM_CLASSES = 81
_NEGPOS_RATIO = 3
_BBOX_THRESH = 0.5
_STDS = (0.1, 0.1, 0.2, 0.2)


def _match_kernel(num_ref, priors_ref, gt_ref, loc_ref, conf_out_ref, lossl_ref):
    b = pl.program_id(0)
    G = gt_ref.shape[1]
    P = priors_ref.shape[1]

    px1 = priors_ref[0:1, :]
    py1 = priors_ref[1:2, :]
    px2 = priors_ref[2:3, :]
    py2 = priors_ref[3:4, :]

    gt = gt_ref[0]
    gx1 = gt[:, 0:1]
    gy1 = gt[:, 1:2]
    gx2 = gt[:, 2:3]
    gy2 = gt[:, 3:4]
    glab = gt[:, 4:5]

    iw = jnp.clip(jnp.minimum(px2, gx2) - jnp.maximum(px1, gx1) + 1.0, 0.0)
    ih = jnp.clip(jnp.minimum(py2, gy2) - jnp.maximum(py1, gy1) + 1.0, 0.0)
    area_p = (px2 - px1 + 1.0) * (py2 - py1 + 1.0)
    area_g = (gx2 - gx1 + 1.0) * (gy2 - gy1 + 1.0)
    inter = iw * ih
    ov = inter / (area_p + area_g - inter)

    giota = jax.lax.broadcasted_iota(jnp.int32, (G, P), 0)
    piota = jax.lax.broadcasted_iota(jnp.int32, (G, P), 1)

    bto = jnp.max(ov, axis=0, keepdims=True)
    bti = jnp.min(jnp.where(ov == bto, giota, G), axis=0, keepdims=True)
    bpo = jnp.max(ov, axis=1, keepdims=True)
    bpi = jnp.min(jnp.where(ov == bpo, piota, P), axis=1, keepdims=True)

    nb = num_ref[b]
    valid = jax.lax.broadcasted_iota(jnp.int32, (G, 1), 0) < nb
    forced = (piota == bpi) & valid
    force_g = jnp.max(jnp.where(forced, giota, -1), axis=0, keepdims=True)
    hit = force_g >= 0
    bti = jnp.where(hit, force_g, bti)
    bto = jnp.where(hit, 2.0, bto)

    oh = giota == bti
    zero = jnp.zeros_like(ov)
    mx1 = jnp.sum(jnp.where(oh, gx1 + zero, 0.0), axis=0, keepdims=True)
    my1 = jnp.sum(jnp.where(oh, gy1 + zero, 0.0), axis=0, keepdims=True)
    mx2 = jnp.sum(jnp.where(oh, gx2 + zero, 0.0), axis=0, keepdims=True)
    my2 = jnp.sum(jnp.where(oh, gy2 + zero, 0.0), axis=0, keepdims=True)
    mlab = jnp.sum(jnp.where(oh, glab + zero, 0.0), axis=0, keepdims=True)

    conf_t = jnp.where(bto < _BBOX_THRESH, 0.0, mlab).astype(jnp.int32)
    conf_out_ref[0] = conf_t

    pw = px2 - px1 + 1.0
    ph = py2 - py1 + 1.0
    pcx = px1 + 0.5 * pw
    pcy = py1 + 0.5 * ph
    gw = mx2 - mx1 + 1.0
    gh = my2 - my1 + 1.0
    gcx = mx1 + 0.5 * gw
    gcy = my1 + 0.5 * gh
    enc = (
        ((gcx - pcx) / pw) / _STDS[0],
        ((gcy - pcy) / ph) / _STDS[1],
        jnp.log(gw / pw) / _STDS[2],
        jnp.log(gh / ph) / _STDS[3],
    )
    loc = loc_ref[0]
    s = jnp.zeros_like(px1)
    for k in range(4):
        d = loc[k : k + 1, :] - enc[k]
        ad = jnp.abs(d)
        s = s + jnp.where(ad < 1.0, 0.5 * d * d, ad - 0.5)
    pos = conf_t > 0
    lossl_ref[b] = jnp.sum(jnp.where(pos, s, 0.0))


def _ce_kernel(conf_ref, ct_ref, ce_ref):
    x = conf_ref[0]
    ct = ct_ref[0]
    m = jnp.max(x, axis=1, keepdims=True)
    lse = jnp.log(jnp.sum(jnp.exp(x - m), axis=1, keepdims=True)) + m
    lane = jax.lax.broadcasted_iota(jnp.int32, x.shape, 1)
    gathered = jnp.sum(jnp.where(lane == ct, x, 0.0), axis=1, keepdims=True)
    ce_ref[0] = lse - gathered


def _final_kernel(lossl_ref, ce_ref, ct_ref, out_ref):
    ce = ce_ref[...]
    pos = ct_ref[...] > 0
    B, P = ce.shape

    num_pos = jnp.sum(pos.astype(jnp.int32), axis=1, keepdims=True)
    k = jnp.minimum(_NEGPOS_RATIO * num_pos, P - 1)
    sum_pos_ce = jnp.sum(jnp.where(pos, ce, 0.0))

    lc = jnp.where(pos, 0.0, ce)
    bits = jax.lax.bitcast_convert_type(lc, jnp.int32)

    def body(_, lh):
        lo, hi = lh
        mid = lo + jax.lax.div(hi - lo, 2)
        cnt = jnp.sum((bits >= mid).astype(jnp.int32), axis=1, keepdims=True)
        take = cnt >= k
        return jnp.where(take, mid, lo), jnp.where(take, hi, mid)

    lo0 = jnp.zeros_like(k)
    hi0 = jnp.full_like(k, jnp.int32(0x7F800000))
    tbits, _ = jax.lax.fori_loop(0, 31, body, (lo0, hi0))

    t = jax.lax.bitcast_convert_type(tbits, jnp.float32)
    gt_mask = bits > tbits
    cnt_gt = jnp.sum(gt_mask.astype(jnp.int32), axis=1, keepdims=True)
    sum_gt = jnp.sum(jnp.where(gt_mask, lc, 0.0), axis=1, keepdims=True)
    topk = sum_gt + (k - cnt_gt).astype(jnp.float32) * t

    loss_c = sum_pos_ce + jnp.sum(topk)
    n = jnp.sum(num_pos).astype(jnp.float32)

    lossl = jnp.float32(0.0)
    for b in range(B):
        lossl = lossl + lossl_ref[b]
    out_ref[0] = lossl / n
    out_ref[1] = loss_c / n


@jax.jit
def kernel(loc_data, conf_data, priors, gt_bboxes, num_boxes):
    B, P, C = conf_data.shape
    G = gt_bboxes.shape[1]
    priors_t = priors.T
    loc_t = loc_data.transpose(0, 2, 1)

    conf_t_raw, lossl = pl.pallas_call(
        _match_kernel,
        grid=(B,),
        in_specs=[
            pl.BlockSpec(memory_space=pltpu.SMEM),
            pl.BlockSpec((4, P), lambda b: (0, 0)),
            pl.BlockSpec((1, G, 5), lambda b: (b, 0, 0)),
            pl.BlockSpec((1, 4, P), lambda b: (b, 0, 0)),
        ],
        out_specs=[
            pl.BlockSpec((1, 1, P), lambda b: (b, 0, 0)),
            pl.BlockSpec(memory_space=pltpu.SMEM),
        ],
        out_shape=[
            jax.ShapeDtypeStruct((B, 1, P), jnp.int32),
            jax.ShapeDtypeStruct((B,), jnp.float32),
        ],
    )(num_boxes, priors_t, gt_bboxes, loc_t)

    ct_col = conf_t_raw.reshape(B, P, 1)
    TP = 512
    nt = pl.cdiv(P, TP)
    ce = pl.pallas_call(
        _ce_kernel,
        grid=(B, nt),
        in_specs=[
            pl.BlockSpec((1, TP, C), lambda b, t: (b, t, 0)),
            pl.BlockSpec((1, TP, 1), lambda b, t: (b, t, 0)),
        ],
        out_specs=pl.BlockSpec((1, TP, 1), lambda b, t: (b, t, 0)),
        out_shape=jax.ShapeDtypeStruct((B, P, 1), jnp.float32),
    )(conf_data, ct_col)

    out = pl.pallas_call(
        _final_kernel,
        in_specs=[
            pl.BlockSpec(memory_space=pltpu.SMEM),
            pl.BlockSpec((B, P), lambda: (0, 0)),
            pl.BlockSpec((B, P), lambda: (0, 0)),
        ],
        out_specs=pl.BlockSpec(memory_space=pltpu.SMEM),
        out_shape=jax.ShapeDtypeStruct((2,), jnp.float32),
    )(lossl, ce.reshape(B, P), conf_t_raw.reshape(B, P))

    return (out[0], out[1])
